# baseline (speedup 1.0000x reference)
import functools

import jax
import jax.numpy as jnp
from jax import lax
from jax.experimental import pallas as pl
from jax.experimental.pallas import tpu as pltpu
from jax.experimental.pallas import tpu_sc as plsc

NUM_ROWS = 1000000
DIM = 32
BATCH = 16384
FIELDS = 26
TOT = BATCH * FIELDS

NC = 2
NS = 16
NW = NC * NS
ROWS_PER_W = TOT // NW
CHUNK = 128
NCHUNK = ROWS_PER_W // CHUNK
NBUF = 4

assert ROWS_PER_W * NW == TOT
assert NCHUNK * CHUNK == ROWS_PER_W
assert NCHUNK % NBUF == 0


_SCALE_BLK = 25000


def _maxabs_body(w_ref, s_ref):
    i = pl.program_id(0)
    m = jnp.max(jnp.abs(w_ref[...]))

    @pl.when(i == 0)
    def _():
        s_ref[0, 0] = m

    @pl.when(i != 0)
    def _():
        s_ref[0, 0] = jnp.maximum(s_ref[0, 0], m)


def _maxabs(weight):
    return pl.pallas_call(
        _maxabs_body,
        grid=(NUM_ROWS // _SCALE_BLK,),
        in_specs=[pl.BlockSpec((_SCALE_BLK, DIM), lambda i: (i, 0))],
        out_specs=pl.BlockSpec(memory_space=pltpu.SMEM),
        out_shape=jax.ShapeDtypeStruct((1, 1), jnp.float32),
    )(weight)


def _gather_body(w_hbm, idx_hbm, out_hbm, idx_v, rows_v, sems):
    wid = lax.axis_index("s") * NC + lax.axis_index("c")
    base = wid * ROWS_PER_W
    pltpu.sync_copy(idx_hbm.at[wid], idx_v)

    for b in range(NBUF):
        pltpu.async_copy(w_hbm.at[idx_v.at[b]], rows_v.at[b], sems.at[b])

    def outer(i, carry):
        for b in range(NBUF):
            g = i * NBUF + b
            pltpu.make_async_copy(
                w_hbm.at[idx_v.at[g]], rows_v.at[b], sems.at[b]
            ).wait()
            pltpu.sync_copy(
                rows_v.at[b], out_hbm.at[pl.ds(base + g * CHUNK, CHUNK)]
            )

            @pl.when(g + NBUF < NCHUNK)
            def _():
                pltpu.async_copy(
                    w_hbm.at[idx_v.at[g + NBUF]], rows_v.at[b], sems.at[b]
                )
        return carry

    lax.fori_loop(0, NCHUNK // NBUF, outer, 0)


@functools.partial(
    pl.kernel,
    mesh=plsc.VectorSubcoreMesh(core_axis_name="c", subcore_axis_name="s"),
    compiler_params=pltpu.CompilerParams(use_tc_tiling_on_sc=False),
    out_type=jax.ShapeDtypeStruct((TOT, DIM), jnp.float32),
    scratch_types=[
        pltpu.VMEM((NCHUNK, CHUNK), jnp.int32),
        pltpu.VMEM((NBUF, CHUNK, DIM), jnp.float32),
        pltpu.SemaphoreType.DMA((NBUF,)),
    ],
)
def _gather(w_hbm, idx_hbm, out_hbm, idx_v, rows_v, sems):
    _gather_body(w_hbm, idx_hbm, out_hbm, idx_v, rows_v, sems)


_Q_BLK = 8192


def _quant_body(m_ref, raw_ref, o_ref):
    s = m_ref[0, 0] / 127.0
    s = jnp.where(s == 0.0, 1.0, s)
    w = raw_ref[...]
    o_ref[...] = jnp.clip(jnp.round(w / s), -127.0, 127.0) * s


def _quant(maxabs, raw):
    return pl.pallas_call(
        _quant_body,
        grid=(TOT // _Q_BLK,),
        in_specs=[
            pl.BlockSpec(memory_space=pltpu.SMEM),
            pl.BlockSpec((_Q_BLK, DIM), lambda i: (i, 0)),
        ],
        out_specs=pl.BlockSpec((_Q_BLK, DIM), lambda i: (i, 0)),
        out_shape=jax.ShapeDtypeStruct((TOT, DIM), jnp.float32),
    )(maxabs, raw)


def kernel(x, weight):
    xi = x.astype(jnp.int32).reshape(NW, NCHUNK, CHUNK)
    maxabs = _maxabs(weight)
    raw = _gather(weight, xi)
    out = _quant(maxabs, raw)
    return out.reshape(BATCH, FIELDS, DIM)


# speedup vs baseline: 1.5561x; 1.5561x over previous
import functools

import jax
import jax.numpy as jnp
from jax import lax
from jax.experimental import pallas as pl
from jax.experimental.pallas import tpu as pltpu
from jax.experimental.pallas import tpu_sc as plsc

NUM_ROWS = 1000000
DIM = 32
BATCH = 16384
FIELDS = 26
TOT = BATCH * FIELDS
W128 = NUM_ROWS * DIM // 128

NC = 2
NS = 16
NW = NC * NS
ROWS_PER_W = TOT // NW
CHUNK = 128
NCHUNK = ROWS_PER_W // CHUNK
NBUF = 4


_MA_BLK = 10000


def _maxabs_body(w_ref, s_ref):
    i = pl.program_id(0)
    m = jnp.max(jnp.abs(w_ref[...]))

    @pl.when(i == 0)
    def _():
        s_ref[0, 0] = m

    @pl.when(i != 0)
    def _():
        s_ref[0, 0] = jnp.maximum(s_ref[0, 0], m)


def _maxabs(w128):
    return pl.pallas_call(
        _maxabs_body,
        grid=(W128 // _MA_BLK,),
        in_specs=[pl.BlockSpec((_MA_BLK, 128), lambda i: (i, 0))],
        out_specs=pl.BlockSpec(memory_space=pltpu.SMEM),
        out_shape=jax.ShapeDtypeStruct((1, 1), jnp.float32),
    )(w128)


_Q_BLK = 10000


def _quant_body(m_ref, w_ref, o_ref):
    s = m_ref[0, 0] / 127.0
    s = jnp.where(s == 0.0, 1.0, s)
    w = w_ref[...]
    o_ref[...] = jnp.clip(jnp.round(w / s), -127.0, 127.0) * s


def _quant(maxabs, w128):
    return pl.pallas_call(
        _quant_body,
        grid=(W128 // _Q_BLK,),
        in_specs=[
            pl.BlockSpec(memory_space=pltpu.SMEM),
            pl.BlockSpec((_Q_BLK, 128), lambda i: (i, 0)),
        ],
        out_specs=pl.BlockSpec((_Q_BLK, 128), lambda i: (i, 0)),
        out_shape=jax.ShapeDtypeStruct((W128, 128), jnp.float32),
    )(maxabs, w128)


def _gather_body(w_hbm, idx_hbm, out_hbm, idx_v, rows_v, sems):
    wid = lax.axis_index("s") * NC + lax.axis_index("c")
    base = wid * ROWS_PER_W
    pltpu.sync_copy(idx_hbm.at[wid], idx_v)

    for b in range(NBUF):
        pltpu.async_copy(
            w_hbm.at[idx_v.at[pl.ds(b * CHUNK, CHUNK)]], rows_v.at[b], sems.at[b]
        )

    def outer(i, carry):
        for b in range(NBUF):
            g = i * NBUF + b
            pltpu.make_async_copy(
                w_hbm.at[idx_v.at[pl.ds(g * CHUNK, CHUNK)]],
                rows_v.at[b],
                sems.at[b],
            ).wait()
            pltpu.sync_copy(
                rows_v.at[b], out_hbm.at[pl.ds(base + g * CHUNK, CHUNK)]
            )

            @pl.when(g + NBUF < NCHUNK)
            def _():
                pltpu.async_copy(
                    w_hbm.at[idx_v.at[pl.ds((g + NBUF) * CHUNK, CHUNK)]],
                    rows_v.at[b],
                    sems.at[b],
                )
        return carry

    lax.fori_loop(0, NCHUNK // NBUF, outer, 0)


@functools.partial(
    pl.kernel,
    mesh=plsc.VectorSubcoreMesh(core_axis_name="c", subcore_axis_name="s"),
    compiler_params=pltpu.CompilerParams(use_tc_tiling_on_sc=False),
    out_type=jax.ShapeDtypeStruct((TOT, DIM), jnp.float32),
    scratch_types=[
        pltpu.VMEM((ROWS_PER_W,), jnp.int32),
        pltpu.VMEM((NBUF, CHUNK, DIM), jnp.float32),
        pltpu.SemaphoreType.DMA((NBUF,)),
    ],
)
def _gather(w_hbm, idx_hbm, out_hbm, idx_v, rows_v, sems):
    _gather_body(w_hbm, idx_hbm, out_hbm, idx_v, rows_v, sems)


def kernel(x, weight):
    w128 = weight.reshape(W128, 128)
    maxabs = _maxabs(w128)
    wq = _quant(maxabs, w128)
    xi = x.astype(jnp.int32).reshape(NW, ROWS_PER_W)
    out = _gather(wq.reshape(NUM_ROWS, DIM), xi)
    return out.reshape(BATCH, FIELDS, DIM)
